# Initial kernel scaffold; baseline (speedup 1.0000x reference)
import jax
import jax.numpy as jnp
from jax.experimental import pallas as pl


def kernel(feats, edge_index, edge_attr, Wq, bq, Wk, bk, Wv, bv, Wo, bo):
    raise NotImplementedError("write your pallas kernel here")


# speedup vs baseline: 18.5182x; 18.5182x over previous
import jax
import jax.numpy as jnp
from jax import lax
from jax.experimental import pallas as pl
from jax.experimental.pallas import tpu as pltpu
from jax.experimental.pallas import tpu_sc as plsc
import functools

N = 10000
DEG = 32
D = 128
H = 8
HD = D // H
E = N * DEG

BP = 1000


def _proj_body(x_ref, wq_ref, wk_ref, wv_ref, bq_ref, bk_ref, bv_ref,
               q_ref, k_ref, v_ref):
  x = x_ref[...]
  q_ref[...] = jnp.dot(x, wq_ref[...], preferred_element_type=jnp.float32) + bq_ref[...]
  k_ref[...] = jnp.dot(x, wk_ref[...], preferred_element_type=jnp.float32) + bk_ref[...]
  v_ref[...] = jnp.dot(x, wv_ref[...], preferred_element_type=jnp.float32) + bv_ref[...]


def _project(feats, wqt, wkt, wvt, bq2, bk2, bv2):
  full = lambda i: (0, 0)
  blk = lambda i: (i, 0)
  return pl.pallas_call(
      _proj_body,
      grid=(N // BP,),
      in_specs=[
          pl.BlockSpec((BP, D), blk),
          pl.BlockSpec((D, D), full),
          pl.BlockSpec((D, D), full),
          pl.BlockSpec((D, D), full),
          pl.BlockSpec((1, D), full),
          pl.BlockSpec((1, D), full),
          pl.BlockSpec((1, D), full),
      ],
      out_specs=[pl.BlockSpec((BP, D), blk)] * 3,
      out_shape=[jax.ShapeDtypeStruct((N, D), jnp.float32)] * 3,
  )(feats, wqt, wkt, wvt, bq2, bk2, bv2)


NC = 2
NS = 16
NW = NC * NS
PER_W = E // NW
CHUNK = 80
NCHUNK = PER_W // CHUNK


def _gather_body(k_hbm, v_hbm, src_hbm, kg_hbm, vg_hbm,
                 idx_v, krows, vrows, sem_k, sem_v):
  wid = lax.axis_index("s") * NC + lax.axis_index("c")
  base0 = wid * PER_W

  def body(t, carry):
    base = base0 + t * CHUNK
    pltpu.sync_copy(src_hbm.at[pl.ds(base, CHUNK)], idx_v)
    ck = pltpu.async_copy(k_hbm.at[idx_v], krows, sem_k)
    cv = pltpu.async_copy(v_hbm.at[idx_v], vrows, sem_v)
    ck.wait()
    cv.wait()
    pltpu.sync_copy(krows, kg_hbm.at[pl.ds(base, CHUNK)])
    pltpu.sync_copy(vrows, vg_hbm.at[pl.ds(base, CHUNK)])
    return carry

  lax.fori_loop(0, NCHUNK, body, 0)


def _gather(k, v, src):
  mesh = plsc.VectorSubcoreMesh(core_axis_name="c", subcore_axis_name="s")
  fn = functools.partial(
      pl.kernel,
      mesh=mesh,
      out_type=[
          jax.ShapeDtypeStruct((E, D), jnp.float32),
          jax.ShapeDtypeStruct((E, D), jnp.float32),
      ],
      scratch_types=[
          pltpu.VMEM((CHUNK,), jnp.int32),
          pltpu.VMEM((CHUNK, D), jnp.float32),
          pltpu.VMEM((CHUNK, D), jnp.float32),
          pltpu.SemaphoreType.DMA,
          pltpu.SemaphoreType.DMA,
      ],
  )(_gather_body)
  return fn(k, v, src)


BA = 200
ISCALE = 1.0 / (HD ** 0.5)


def _attn_body(q_ref, kg_ref, vg_ref, hm_ref, hmt_ref, wo_ref, bo_ref, o_ref):
  q = q_ref[...]
  kg = kg_ref[...]
  vg = vg_ref[...]
  prod = (kg.reshape(BA, DEG, D) * q[:, None, :]).reshape(BA * DEG, D)
  sim = jnp.dot(prod, hm_ref[...], preferred_element_type=jnp.float32)
  sim = (sim * ISCALE).reshape(BA, DEG, D)
  m = jnp.max(sim, axis=1, keepdims=True)
  p = jnp.exp(sim - m)
  s = jnp.sum(p, axis=1, keepdims=True)
  attn = (p / s).reshape(BA * DEG, D)
  aw = jnp.dot(attn, hmt_ref[...], preferred_element_type=jnp.float32)
  ov = (aw * vg).reshape(BA, DEG, D).sum(axis=1)
  o_ref[...] = jnp.dot(ov, wo_ref[...], preferred_element_type=jnp.float32) + bo_ref[...]


def _attention(q, kg, vg, hm, hmt, wot, bo2):
  full = lambda i: (0, 0)
  return pl.pallas_call(
      _attn_body,
      grid=(N // BA,),
      in_specs=[
          pl.BlockSpec((BA, D), lambda i: (i, 0)),
          pl.BlockSpec((BA * DEG, D), lambda i: (i, 0)),
          pl.BlockSpec((BA * DEG, D), lambda i: (i, 0)),
          pl.BlockSpec((D, D), full),
          pl.BlockSpec((D, D), full),
          pl.BlockSpec((D, D), full),
          pl.BlockSpec((1, D), full),
      ],
      out_specs=pl.BlockSpec((BA, D), lambda i: (i, 0)),
      out_shape=jax.ShapeDtypeStruct((N, D), jnp.float32),
  )(q, kg, vg, hm, hmt, wot, bo2)


def kernel(feats, edge_index, edge_attr, Wq, bq, Wk, bk, Wv, bv, Wo, bo):
  del edge_attr
  q, k, v = _project(feats, Wq.T, Wk.T, Wv.T,
                     bq.reshape(1, D), bk.reshape(1, D), bv.reshape(1, D))
  src = edge_index[:, 0]
  kg, vg = _gather(k, v, src)
  d_ids = jnp.arange(D, dtype=jnp.int32)
  hm = (d_ids[:, None] // HD == d_ids[None, :]).astype(jnp.float32)
  hmt = (d_ids[:, None] == d_ids[None, :] // HD).astype(jnp.float32)
  out = _attention(q, kg, vg, hm, hmt, Wo.T, bo.reshape(1, D))
  return out


# speedup vs baseline: 24.3701x; 1.3160x over previous
import jax
import jax.numpy as jnp
from jax import lax
from jax.experimental import pallas as pl
from jax.experimental.pallas import tpu as pltpu
from jax.experimental.pallas import tpu_sc as plsc
import functools

N = 10000
DEG = 32
D = 128
H = 8
HD = D // H
E = N * DEG

BP = 1000


def _proj_body(x_ref, wq_ref, wk_ref, wv_ref, bq_ref, bk_ref, bv_ref,
               q_ref, k_ref, v_ref):
  x = x_ref[...]
  q_ref[...] = jnp.dot(x, wq_ref[...], preferred_element_type=jnp.float32) + bq_ref[...]
  k_ref[...] = jnp.dot(x, wk_ref[...], preferred_element_type=jnp.float32) + bk_ref[...]
  v_ref[...] = jnp.dot(x, wv_ref[...], preferred_element_type=jnp.float32) + bv_ref[...]


def _project(feats, wqt, wkt, wvt, bq2, bk2, bv2):
  full = lambda i: (0, 0)
  blk = lambda i: (i, 0)
  return pl.pallas_call(
      _proj_body,
      grid=(N // BP,),
      in_specs=[
          pl.BlockSpec((BP, D), blk),
          pl.BlockSpec((D, D), full),
          pl.BlockSpec((D, D), full),
          pl.BlockSpec((D, D), full),
          pl.BlockSpec((1, D), full),
          pl.BlockSpec((1, D), full),
          pl.BlockSpec((1, D), full),
      ],
      out_specs=[pl.BlockSpec((BP, D), blk)] * 3,
      out_shape=[jax.ShapeDtypeStruct((N, D), jnp.float32)] * 3,
  )(feats, wqt, wkt, wvt, bq2, bk2, bv2)


NC = 2
NS = 16
NW = NC * NS
PER_W = E // NW
CHUNK = 80
GRP = 5
NGRP = PER_W // CHUNK // GRP


def _gather_body(k_hbm, v_hbm, src_hbm, kg_hbm, vg_hbm, *scr):
  idx_v = scr[0]
  krows = scr[1:1 + GRP]
  vrows = scr[1 + GRP:1 + 2 * GRP]
  sems = scr[1 + 2 * GRP:]
  sem_gk, sem_gv = sems[0:GRP], sems[GRP:2 * GRP]
  sem_sk, sem_sv = sems[2 * GRP:3 * GRP], sems[3 * GRP:4 * GRP]
  wid = lax.axis_index("s") * NC + lax.axis_index("c")
  base0 = wid * PER_W

  def body(g, carry):
    base = base0 + g * (GRP * CHUNK)
    pltpu.sync_copy(src_hbm.at[pl.ds(base, GRP * CHUNK)], idx_v)
    gk = [pltpu.async_copy(k_hbm.at[idx_v.at[pl.ds(s * CHUNK, CHUNK)]],
                           krows[s], sem_gk[s]) for s in range(GRP)]
    gv = [pltpu.async_copy(v_hbm.at[idx_v.at[pl.ds(s * CHUNK, CHUNK)]],
                           vrows[s], sem_gv[s]) for s in range(GRP)]
    sk, sv = [], []
    for s in range(GRP):
      gk[s].wait()
      sk.append(pltpu.async_copy(
          krows[s], kg_hbm.at[pl.ds(base + s * CHUNK, CHUNK)], sem_sk[s]))
    for s in range(GRP):
      gv[s].wait()
      sv.append(pltpu.async_copy(
          vrows[s], vg_hbm.at[pl.ds(base + s * CHUNK, CHUNK)], sem_sv[s]))
    for s in range(GRP):
      sk[s].wait()
      sv[s].wait()
    return carry

  lax.fori_loop(0, NGRP, body, 0)


def _gather(k, v, src):
  mesh = plsc.VectorSubcoreMesh(core_axis_name="c", subcore_axis_name="s")
  fn = functools.partial(
      pl.kernel,
      mesh=mesh,
      out_type=[
          jax.ShapeDtypeStruct((E, D), jnp.float32),
          jax.ShapeDtypeStruct((E, D), jnp.float32),
      ],
      scratch_types=(
          [pltpu.VMEM((GRP * CHUNK,), jnp.int32)]
          + [pltpu.VMEM((CHUNK, D), jnp.float32)] * (2 * GRP)
          + [pltpu.SemaphoreType.DMA] * (4 * GRP)
      ),
  )(_gather_body)
  return fn(k, v, src)


BA = 200
ISCALE = 1.0 / (HD ** 0.5)


def _attn_body(q_ref, kg_ref, vg_ref, hm_ref, hmt_ref, wo_ref, bo_ref, o_ref):
  q = q_ref[...]
  kg = kg_ref[...]
  vg = vg_ref[...]
  prod = (kg.reshape(BA, DEG, D) * q[:, None, :]).reshape(BA * DEG, D)
  sim = jnp.dot(prod, hm_ref[...], preferred_element_type=jnp.float32)
  sim = (sim * ISCALE).reshape(BA, DEG, D)
  m = jnp.max(sim, axis=1, keepdims=True)
  p = jnp.exp(sim - m)
  s = jnp.sum(p, axis=1, keepdims=True)
  attn = (p / s).reshape(BA * DEG, D)
  aw = jnp.dot(attn, hmt_ref[...], preferred_element_type=jnp.float32)
  ov = (aw * vg).reshape(BA, DEG, D).sum(axis=1)
  o_ref[...] = jnp.dot(ov, wo_ref[...], preferred_element_type=jnp.float32) + bo_ref[...]


def _attention(q, kg, vg, hm, hmt, wot, bo2):
  full = lambda i: (0, 0)
  return pl.pallas_call(
      _attn_body,
      grid=(N // BA,),
      in_specs=[
          pl.BlockSpec((BA, D), lambda i: (i, 0)),
          pl.BlockSpec((BA * DEG, D), lambda i: (i, 0)),
          pl.BlockSpec((BA * DEG, D), lambda i: (i, 0)),
          pl.BlockSpec((D, D), full),
          pl.BlockSpec((D, D), full),
          pl.BlockSpec((D, D), full),
          pl.BlockSpec((1, D), full),
      ],
      out_specs=pl.BlockSpec((BA, D), lambda i: (i, 0)),
      out_shape=jax.ShapeDtypeStruct((N, D), jnp.float32),
  )(q, kg, vg, hm, hmt, wot, bo2)


def kernel(feats, edge_index, edge_attr, Wq, bq, Wk, bk, Wv, bv, Wo, bo):
  del edge_attr
  q, k, v = _project(feats, Wq.T, Wk.T, Wv.T,
                     bq.reshape(1, D), bk.reshape(1, D), bv.reshape(1, D))
  src = edge_index[:, 0]
  kg, vg = _gather(k, v, src)
  d_ids = jnp.arange(D, dtype=jnp.int32)
  hm = (d_ids[:, None] // HD == d_ids[None, :]).astype(jnp.float32)
  hmt = (d_ids[:, None] == d_ids[None, :] // HD).astype(jnp.float32)
  out = _attention(q, kg, vg, hm, hmt, Wo.T, bo.reshape(1, D))
  return out


# speedup vs baseline: 34.2043x; 1.4035x over previous
import jax
import jax.numpy as jnp
from jax import lax
from jax.experimental import pallas as pl
from jax.experimental.pallas import tpu as pltpu
from jax.experimental.pallas import tpu_sc as plsc
import functools

N = 10000
DEG = 32
D = 128
H = 8
HD = D // H
E = N * DEG

BP = 1000


def _proj_body(x_ref, wq_ref, wk_ref, wv_ref, bq_ref, bk_ref, bv_ref,
               q_ref, kv_ref):
  x = x_ref[...]
  q_ref[...] = jnp.dot(x, wq_ref[...], preferred_element_type=jnp.float32) + bq_ref[...]
  k = jnp.dot(x, wk_ref[...], preferred_element_type=jnp.float32) + bk_ref[...]
  v = jnp.dot(x, wv_ref[...], preferred_element_type=jnp.float32) + bv_ref[...]
  kb = lax.bitcast_convert_type(k.astype(jnp.bfloat16), jnp.uint16).astype(jnp.uint32)
  vb = lax.bitcast_convert_type(v.astype(jnp.bfloat16), jnp.uint16).astype(jnp.uint32)
  kv_ref[...] = lax.bitcast_convert_type(kb | (vb << 16), jnp.int32)


def _project(feats, wqt, wkt, wvt, bq2, bk2, bv2):
  full = lambda i: (0, 0)
  blk = lambda i: (i, 0)
  return pl.pallas_call(
      _proj_body,
      grid=(N // BP,),
      in_specs=[
          pl.BlockSpec((BP, D), blk),
          pl.BlockSpec((D, D), full),
          pl.BlockSpec((D, D), full),
          pl.BlockSpec((D, D), full),
          pl.BlockSpec((1, D), full),
          pl.BlockSpec((1, D), full),
          pl.BlockSpec((1, D), full),
      ],
      out_specs=[pl.BlockSpec((BP, D), blk)] * 2,
      out_shape=[
          jax.ShapeDtypeStruct((N, D), jnp.float32),
          jax.ShapeDtypeStruct((N, D), jnp.int32),
      ],
  )(feats, wqt, wkt, wvt, bq2, bk2, bv2)


NC = 2
NS = 16
NW = NC * NS
PER_W = E // NW
CHUNK = 80
GRP = 5
NGRP = PER_W // CHUNK // GRP


def _gather_body(kv_hbm, src_hbm, kvg_hbm, *scr):
  idx_v = scr[0]
  rows = scr[1:1 + GRP]
  sems = scr[1 + GRP:]
  sem_g, sem_s = sems[0:GRP], sems[GRP:2 * GRP]
  wid = lax.axis_index("s") * NC + lax.axis_index("c")
  base0 = wid * PER_W

  def body(g, carry):
    base = base0 + g * (GRP * CHUNK)
    pltpu.sync_copy(src_hbm.at[pl.ds(base, GRP * CHUNK)], idx_v)
    gs = [pltpu.async_copy(kv_hbm.at[idx_v.at[pl.ds(s * CHUNK, CHUNK)]],
                           rows[s], sem_g[s]) for s in range(GRP)]
    st = []
    for s in range(GRP):
      gs[s].wait()
      st.append(pltpu.async_copy(
          rows[s], kvg_hbm.at[pl.ds(base + s * CHUNK, CHUNK)], sem_s[s]))
    for s in range(GRP):
      st[s].wait()
    return carry

  lax.fori_loop(0, NGRP, body, 0)


def _gather(kv, src):
  mesh = plsc.VectorSubcoreMesh(core_axis_name="c", subcore_axis_name="s")
  fn = functools.partial(
      pl.kernel,
      mesh=mesh,
      out_type=jax.ShapeDtypeStruct((E, D), jnp.int32),
      scratch_types=(
          [pltpu.VMEM((GRP * CHUNK,), jnp.int32)]
          + [pltpu.VMEM((CHUNK, D), jnp.int32)] * GRP
          + [pltpu.SemaphoreType.DMA] * (2 * GRP)
      ),
  )(_gather_body)
  return fn(kv, src)


BA = 200
ISCALE = 1.0 / (HD ** 0.5)


def _attn_body(q_ref, kvg_ref, hm_ref, hmt_ref, wo_ref, bo_ref, o_ref):
  q = q_ref[...]
  kvg = kvg_ref[...]
  kg = lax.bitcast_convert_type(kvg << 16, jnp.float32)
  vg = lax.bitcast_convert_type(kvg & jnp.int32(-65536), jnp.float32)
  prod = (kg.reshape(BA, DEG, D) * q[:, None, :]).reshape(BA * DEG, D)
  sim = jnp.dot(prod, hm_ref[...], preferred_element_type=jnp.float32)
  sim = (sim * ISCALE).reshape(BA, DEG, D)
  m = jnp.max(sim, axis=1, keepdims=True)
  p = jnp.exp(sim - m)
  s = jnp.sum(p, axis=1, keepdims=True)
  attn = (p / s).reshape(BA * DEG, D)
  aw = jnp.dot(attn, hmt_ref[...], preferred_element_type=jnp.float32)
  ov = (aw * vg).reshape(BA, DEG, D).sum(axis=1)
  o_ref[...] = jnp.dot(ov, wo_ref[...], preferred_element_type=jnp.float32) + bo_ref[...]


def _attention(q, kvg, hm, hmt, wot, bo2):
  full = lambda i: (0, 0)
  return pl.pallas_call(
      _attn_body,
      grid=(N // BA,),
      in_specs=[
          pl.BlockSpec((BA, D), lambda i: (i, 0)),
          pl.BlockSpec((BA * DEG, D), lambda i: (i, 0)),
          pl.BlockSpec((D, D), full),
          pl.BlockSpec((D, D), full),
          pl.BlockSpec((D, D), full),
          pl.BlockSpec((1, D), full),
      ],
      out_specs=pl.BlockSpec((BA, D), lambda i: (i, 0)),
      out_shape=jax.ShapeDtypeStruct((N, D), jnp.float32),
  )(q, kvg, hm, hmt, wot, bo2)


def kernel(feats, edge_index, edge_attr, Wq, bq, Wk, bk, Wv, bv, Wo, bo):
  del edge_attr
  q, kv = _project(feats, Wq.T, Wk.T, Wv.T,
                   bq.reshape(1, D), bk.reshape(1, D), bv.reshape(1, D))
  src = edge_index[:, 0]
  kvg = _gather(kv, src)
  d_ids = jnp.arange(D, dtype=jnp.int32)
  hm = (d_ids[:, None] // HD == d_ids[None, :]).astype(jnp.float32)
  hmt = (d_ids[:, None] == d_ids[None, :] // HD).astype(jnp.float32)
  out = _attention(q, kvg, hm, hmt, Wo.T, bo.reshape(1, D))
  return out


# speedup vs baseline: 35.9679x; 1.0516x over previous
import jax
import jax.numpy as jnp
from jax import lax
from jax.experimental import pallas as pl
from jax.experimental.pallas import tpu as pltpu
from jax.experimental.pallas import tpu_sc as plsc
import functools

N = 10000
DEG = 32
D = 128
H = 8
HD = D // H
E = N * DEG

BP = 1000


def _proj_body(x_ref, wq_ref, wk_ref, wv_ref, bq_ref, bk_ref, bv_ref,
               q_ref, kv_ref):
  x = x_ref[...]
  q_ref[...] = jnp.dot(x, wq_ref[...], preferred_element_type=jnp.float32) + bq_ref[...]
  k = jnp.dot(x, wk_ref[...], preferred_element_type=jnp.float32) + bk_ref[...]
  v = jnp.dot(x, wv_ref[...], preferred_element_type=jnp.float32) + bv_ref[...]
  kb = lax.bitcast_convert_type(k.astype(jnp.bfloat16), jnp.uint16).astype(jnp.uint32)
  vb = lax.bitcast_convert_type(v.astype(jnp.bfloat16), jnp.uint16).astype(jnp.uint32)
  kv_ref[...] = lax.bitcast_convert_type(kb | (vb << 16), jnp.int32)


def _project(feats, wqt, wkt, wvt, bq2, bk2, bv2):
  full = lambda i: (0, 0)
  blk = lambda i: (i, 0)
  return pl.pallas_call(
      _proj_body,
      grid=(N // BP,),
      in_specs=[
          pl.BlockSpec((BP, D), blk),
          pl.BlockSpec((D, D), full),
          pl.BlockSpec((D, D), full),
          pl.BlockSpec((D, D), full),
          pl.BlockSpec((1, D), full),
          pl.BlockSpec((1, D), full),
          pl.BlockSpec((1, D), full),
      ],
      out_specs=[pl.BlockSpec((BP, D), blk)] * 2,
      out_shape=[
          jax.ShapeDtypeStruct((N, D), jnp.float32),
          jax.ShapeDtypeStruct((N, D), jnp.int32),
      ],
  )(feats, wqt, wkt, wvt, bq2, bk2, bv2)


NC = 2
NS = 16
NW = NC * NS
PER_W = E // NW
CHUNK = 40
NSLOT = 10
NGRP = PER_W // CHUNK // NSLOT


def _gather_body(kv_hbm, src_hbm, kvg_hbm, *scr):
  idx_all = scr[0]
  rows = scr[1:1 + NSLOT]
  sems = scr[1 + NSLOT:]
  sem_g, sem_s = sems[0:NSLOT], sems[NSLOT:2 * NSLOT]
  wid = lax.axis_index("s") * NC + lax.axis_index("c")
  base0 = wid * PER_W

  pltpu.sync_copy(src_hbm.at[pl.ds(base0, PER_W)], idx_all)

  def fire(slot, chunk):
    pltpu.async_copy(
        kv_hbm.at[idx_all.at[pl.ds(chunk * CHUNK, CHUNK)]],
        rows[slot], sem_g[slot])

  def store(slot, chunk):
    pltpu.async_copy(
        rows[slot], kvg_hbm.at[pl.ds(base0 + chunk * CHUNK, CHUNK)],
        sem_s[slot])

  def drain_gather(slot, chunk):
    pltpu.make_async_copy(
        kv_hbm.at[idx_all.at[pl.ds(chunk * CHUNK, CHUNK)]],
        rows[slot], sem_g[slot]).wait()

  def drain_store(slot, chunk):
    pltpu.make_async_copy(
        rows[slot], kvg_hbm.at[pl.ds(base0 + chunk * CHUNK, CHUNK)],
        sem_s[slot]).wait()

  for s in range(NSLOT):
    fire(s, s)
  for s in range(NSLOT):
    drain_gather(s, s)
    store(s, s)

  def body(j, carry):
    c0 = j * NSLOT
    for s in range(NSLOT):
      drain_store(s, c0 + s - NSLOT)
      fire(s, c0 + s)
    for s in range(NSLOT):
      drain_gather(s, c0 + s)
      store(s, c0 + s)
    return carry

  lax.fori_loop(1, NGRP, body, 0)
  for s in range(NSLOT):
    drain_store(s, (NGRP - 1) * NSLOT + s)


def _gather(kv, src):
  mesh = plsc.VectorSubcoreMesh(core_axis_name="c", subcore_axis_name="s")
  fn = functools.partial(
      pl.kernel,
      mesh=mesh,
      out_type=jax.ShapeDtypeStruct((E, D), jnp.int32),
      scratch_types=(
          [pltpu.VMEM((PER_W,), jnp.int32)]
          + [pltpu.VMEM((CHUNK, D), jnp.int32)] * NSLOT
          + [pltpu.SemaphoreType.DMA] * (2 * NSLOT)
      ),
  )(_gather_body)
  return fn(kv, src)


BA = 200
ISCALE = 1.0 / (HD ** 0.5)


def _attn_body(q_ref, kvg_ref, hm_ref, hmt_ref, wo_ref, bo_ref, o_ref):
  q = q_ref[...]
  kvg = kvg_ref[...]
  kg = lax.bitcast_convert_type(kvg << 16, jnp.float32)
  vg = lax.bitcast_convert_type(kvg & jnp.int32(-65536), jnp.float32)
  prod = (kg.reshape(BA, DEG, D) * q[:, None, :]).reshape(BA * DEG, D)
  sim = jnp.dot(prod, hm_ref[...], preferred_element_type=jnp.float32)
  sim = (sim * ISCALE).reshape(BA, DEG, D)
  m = jnp.max(sim, axis=1, keepdims=True)
  p = jnp.exp(sim - m)
  s = jnp.sum(p, axis=1, keepdims=True)
  attn = (p / s).reshape(BA * DEG, D)
  aw = jnp.dot(attn, hmt_ref[...], preferred_element_type=jnp.float32)
  ov = (aw * vg).reshape(BA, DEG, D).sum(axis=1)
  o_ref[...] = jnp.dot(ov, wo_ref[...], preferred_element_type=jnp.float32) + bo_ref[...]


def _attention(q, kvg, hm, hmt, wot, bo2):
  full = lambda i: (0, 0)
  return pl.pallas_call(
      _attn_body,
      grid=(N // BA,),
      in_specs=[
          pl.BlockSpec((BA, D), lambda i: (i, 0)),
          pl.BlockSpec((BA * DEG, D), lambda i: (i, 0)),
          pl.BlockSpec((D, D), full),
          pl.BlockSpec((D, D), full),
          pl.BlockSpec((D, D), full),
          pl.BlockSpec((1, D), full),
      ],
      out_specs=pl.BlockSpec((BA, D), lambda i: (i, 0)),
      out_shape=jax.ShapeDtypeStruct((N, D), jnp.float32),
  )(q, kvg, hm, hmt, wot, bo2)


def kernel(feats, edge_index, edge_attr, Wq, bq, Wk, bk, Wv, bv, Wo, bo):
  del edge_attr
  q, kv = _project(feats, Wq.T, Wk.T, Wv.T,
                   bq.reshape(1, D), bk.reshape(1, D), bv.reshape(1, D))
  src = edge_index[:, 0]
  kvg = _gather(kv, src)
  d_ids = jnp.arange(D, dtype=jnp.int32)
  hm = (d_ids[:, None] // HD == d_ids[None, :]).astype(jnp.float32)
  hmt = (d_ids[:, None] == d_ids[None, :] // HD).astype(jnp.float32)
  out = _attention(q, kvg, hm, hmt, Wo.T, bo.reshape(1, D))
  return out


# speedup vs baseline: 38.2004x; 1.0621x over previous
import jax
import jax.numpy as jnp
from jax import lax
from jax.experimental import pallas as pl
from jax.experimental.pallas import tpu as pltpu
from jax.experimental.pallas import tpu_sc as plsc
import functools

N = 10000
DEG = 32
D = 128
H = 8
HD = D // H
E = N * DEG

BP = 1000


def _proj_body(x_ref, wq_ref, wk_ref, wv_ref, bq_ref, bk_ref, bv_ref,
               q_ref, kv_ref):
  x = x_ref[...]
  q_ref[...] = jnp.dot(x, wq_ref[...], preferred_element_type=jnp.float32) + bq_ref[...]
  k = jnp.dot(x, wk_ref[...], preferred_element_type=jnp.float32) + bk_ref[...]
  v = jnp.dot(x, wv_ref[...], preferred_element_type=jnp.float32) + bv_ref[...]
  kb = lax.bitcast_convert_type(k.astype(jnp.bfloat16), jnp.uint16).astype(jnp.uint32)
  vb = lax.bitcast_convert_type(v.astype(jnp.bfloat16), jnp.uint16).astype(jnp.uint32)
  kv_ref[...] = lax.bitcast_convert_type(kb | (vb << 16), jnp.int32)


def _project(feats, wqt, wkt, wvt, bq2, bk2, bv2):
  full = lambda i: (0, 0)
  blk = lambda i: (i, 0)
  return pl.pallas_call(
      _proj_body,
      grid=(N // BP,),
      in_specs=[
          pl.BlockSpec((BP, D), blk),
          pl.BlockSpec((D, D), full),
          pl.BlockSpec((D, D), full),
          pl.BlockSpec((D, D), full),
          pl.BlockSpec((1, D), full),
          pl.BlockSpec((1, D), full),
          pl.BlockSpec((1, D), full),
      ],
      out_specs=[pl.BlockSpec((BP, D), blk)] * 2,
      out_shape=[
          jax.ShapeDtypeStruct((N, D), jnp.float32),
          jax.ShapeDtypeStruct((N, D), jnp.int32),
      ],
  )(feats, wqt, wkt, wvt, bq2, bk2, bv2)


NC = 2
NS = 16
NW = NC * NS
CHUNK = 40


def _make_gather(e_s, nslot):
  per_w = e_s // NW
  ngrp = per_w // CHUNK // nslot
  assert per_w % (CHUNK * nslot) == 0

  def _gather_body(kv_hbm, src_hbm, kvg_hbm, *scr):
    idx_all = scr[0]
    rows = scr[1:1 + nslot]
    sems = scr[1 + nslot:]
    sem_g, sem_s = sems[0:nslot], sems[nslot:2 * nslot]
    wid = lax.axis_index("s") * NC + lax.axis_index("c")
    base0 = wid * per_w

    pltpu.sync_copy(src_hbm.at[pl.ds(base0, per_w)], idx_all)

    def fire(slot, chunk):
      pltpu.async_copy(
          kv_hbm.at[idx_all.at[pl.ds(chunk * CHUNK, CHUNK)]],
          rows[slot], sem_g[slot])

    def store(slot, chunk):
      pltpu.async_copy(
          rows[slot], kvg_hbm.at[pl.ds(base0 + chunk * CHUNK, CHUNK)],
          sem_s[slot])

    def drain_gather(slot, chunk):
      pltpu.make_async_copy(
          kv_hbm.at[idx_all.at[pl.ds(chunk * CHUNK, CHUNK)]],
          rows[slot], sem_g[slot]).wait()

    def drain_store(slot, chunk):
      pltpu.make_async_copy(
          rows[slot], kvg_hbm.at[pl.ds(base0 + chunk * CHUNK, CHUNK)],
          sem_s[slot]).wait()

    for s in range(nslot):
      fire(s, s)
    for s in range(nslot):
      drain_gather(s, s)
      store(s, s)

    def body(j, carry):
      c0 = j * nslot
      for s in range(nslot):
        drain_store(s, c0 + s - nslot)
        fire(s, c0 + s)
      for s in range(nslot):
        drain_gather(s, c0 + s)
        store(s, c0 + s)
      return carry

    lax.fori_loop(1, ngrp, body, 0)
    for s in range(nslot):
      drain_store(s, (ngrp - 1) * nslot + s)

  mesh = plsc.VectorSubcoreMesh(core_axis_name="c", subcore_axis_name="s")
  return functools.partial(
      pl.kernel,
      mesh=mesh,
      out_type=jax.ShapeDtypeStruct((e_s, D), jnp.int32),
      scratch_types=(
          [pltpu.VMEM((per_w,), jnp.int32)]
          + [pltpu.VMEM((CHUNK, D), jnp.int32)] * nslot
          + [pltpu.SemaphoreType.DMA] * (2 * nslot)
      ),
  )(_gather_body)


BA = 200
ISCALE = 1.0 / (HD ** 0.5)
SLICES = 2


def _attn_body(q_ref, kvg_ref, hm_ref, hmt_ref, wo_ref, bo_ref, o_ref):
  q = q_ref[...]
  kvg = kvg_ref[...]
  kg = lax.bitcast_convert_type(kvg << 16, jnp.float32)
  vg = lax.bitcast_convert_type(kvg & jnp.int32(-65536), jnp.float32)
  prod = (kg.reshape(BA, DEG, D) * q[:, None, :]).reshape(BA * DEG, D)
  sim = jnp.dot(prod, hm_ref[...], preferred_element_type=jnp.float32)
  sim = (sim * ISCALE).reshape(BA, DEG, D)
  m = jnp.max(sim, axis=1, keepdims=True)
  p = jnp.exp(sim - m)
  s = jnp.sum(p, axis=1, keepdims=True)
  attn = (p / s).reshape(BA * DEG, D)
  aw = jnp.dot(attn, hmt_ref[...], preferred_element_type=jnp.float32)
  ov = (aw * vg).reshape(BA, DEG, D).sum(axis=1)
  o_ref[...] = jnp.dot(ov, wo_ref[...], preferred_element_type=jnp.float32) + bo_ref[...]


def _attention(q, kvg, hm, hmt, wot, bo2, n_nodes, node_off):
  full = lambda i: (0, 0)
  off_blk = node_off // BA
  return pl.pallas_call(
      _attn_body,
      grid=(n_nodes // BA,),
      in_specs=[
          pl.BlockSpec((BA, D), lambda i: (i + off_blk, 0)),
          pl.BlockSpec((BA * DEG, D), lambda i: (i, 0)),
          pl.BlockSpec((D, D), full),
          pl.BlockSpec((D, D), full),
          pl.BlockSpec((D, D), full),
          pl.BlockSpec((1, D), full),
      ],
      out_specs=pl.BlockSpec((BA, D), lambda i: (i, 0)),
      out_shape=jax.ShapeDtypeStruct((n_nodes, D), jnp.float32),
  )(q, kvg, hm, hmt, wot, bo2)


def kernel(feats, edge_index, edge_attr, Wq, bq, Wk, bk, Wv, bv, Wo, bo):
  del edge_attr
  q, kv = _project(feats, Wq.T, Wk.T, Wv.T,
                   bq.reshape(1, D), bk.reshape(1, D), bv.reshape(1, D))
  src = edge_index[:, 0]
  d_ids = jnp.arange(D, dtype=jnp.int32)
  hm = (d_ids[:, None] // HD == d_ids[None, :]).astype(jnp.float32)
  hmt = (d_ids[:, None] == d_ids[None, :] // HD).astype(jnp.float32)
  wot = Wo.T
  bo2 = bo.reshape(1, D)
  ns = N // SLICES
  es = E // SLICES
  gather = _make_gather(es, 5)
  kvgs = [gather(kv, lax.slice_in_dim(src, i * es, (i + 1) * es))
          for i in range(SLICES)]
  outs = [_attention(q, kvgs[i], hm, hmt, wot, bo2, ns, i * ns)
          for i in range(SLICES)]
  return jnp.concatenate(outs, axis=0)


# speedup vs baseline: 39.0621x; 1.0226x over previous
import jax
import jax.numpy as jnp
from jax import lax
from jax.experimental import pallas as pl
from jax.experimental.pallas import tpu as pltpu
from jax.experimental.pallas import tpu_sc as plsc
import functools

N = 10000
DEG = 32
D = 128
H = 8
HD = D // H
E = N * DEG

BP = 1000


def _proj_body(x_ref, wq_ref, wk_ref, wv_ref, bq_ref, bk_ref, bv_ref,
               q_ref, kv_ref):
  x = x_ref[...]
  q_ref[...] = jnp.dot(x, wq_ref[...], preferred_element_type=jnp.float32) + bq_ref[...]
  k = jnp.dot(x, wk_ref[...], preferred_element_type=jnp.float32) + bk_ref[...]
  v = jnp.dot(x, wv_ref[...], preferred_element_type=jnp.float32) + bv_ref[...]
  kb = lax.bitcast_convert_type(k.astype(jnp.bfloat16), jnp.uint16).astype(jnp.uint32)
  vb = lax.bitcast_convert_type(v.astype(jnp.bfloat16), jnp.uint16).astype(jnp.uint32)
  kv_ref[...] = lax.bitcast_convert_type(kb | (vb << 16), jnp.int32)


def _project(feats, wqt, wkt, wvt, bq2, bk2, bv2):
  full = lambda i: (0, 0)
  blk = lambda i: (i, 0)
  return pl.pallas_call(
      _proj_body,
      grid=(N // BP,),
      in_specs=[
          pl.BlockSpec((BP, D), blk),
          pl.BlockSpec((D, D), full),
          pl.BlockSpec((D, D), full),
          pl.BlockSpec((D, D), full),
          pl.BlockSpec((1, D), full),
          pl.BlockSpec((1, D), full),
          pl.BlockSpec((1, D), full),
      ],
      out_specs=[pl.BlockSpec((BP, D), blk)] * 2,
      out_shape=[
          jax.ShapeDtypeStruct((N, D), jnp.float32),
          jax.ShapeDtypeStruct((N, D), jnp.int32),
      ],
  )(feats, wqt, wkt, wvt, bq2, bk2, bv2)


NC = 2
NS = 16
NW = NC * NS
CHUNK = 40


def _make_gather(e_s, nslot):
  per_w = e_s // NW
  ngrp = per_w // CHUNK // nslot
  assert per_w % (CHUNK * nslot) == 0

  def _gather_body(kv_hbm, src_hbm, kvg_hbm, *scr):
    idx_all = scr[0]
    rows = scr[1:1 + nslot]
    sems = scr[1 + nslot:]
    sem_g, sem_s = sems[0:nslot], sems[nslot:2 * nslot]
    wid = lax.axis_index("s") * NC + lax.axis_index("c")
    base0 = wid * per_w

    pltpu.sync_copy(src_hbm.at[pl.ds(base0, per_w)], idx_all)

    def fire(slot, chunk):
      pltpu.async_copy(
          kv_hbm.at[idx_all.at[pl.ds(chunk * CHUNK, CHUNK)]],
          rows[slot], sem_g[slot])

    def store(slot, chunk):
      pltpu.async_copy(
          rows[slot], kvg_hbm.at[pl.ds(base0 + chunk * CHUNK, CHUNK)],
          sem_s[slot])

    def drain_gather(slot, chunk):
      pltpu.make_async_copy(
          kv_hbm.at[idx_all.at[pl.ds(chunk * CHUNK, CHUNK)]],
          rows[slot], sem_g[slot]).wait()

    def drain_store(slot, chunk):
      pltpu.make_async_copy(
          rows[slot], kvg_hbm.at[pl.ds(base0 + chunk * CHUNK, CHUNK)],
          sem_s[slot]).wait()

    for s in range(nslot):
      fire(s, s)
    for s in range(nslot):
      drain_gather(s, s)
      store(s, s)

    def body(j, carry):
      c0 = j * nslot
      for s in range(nslot):
        drain_store(s, c0 + s - nslot)
        fire(s, c0 + s)
      for s in range(nslot):
        drain_gather(s, c0 + s)
        store(s, c0 + s)
      return carry

    lax.fori_loop(1, ngrp, body, 0)
    for s in range(nslot):
      drain_store(s, (ngrp - 1) * nslot + s)

  mesh = plsc.VectorSubcoreMesh(core_axis_name="c", subcore_axis_name="s")
  return functools.partial(
      pl.kernel,
      mesh=mesh,
      out_type=jax.ShapeDtypeStruct((e_s, D), jnp.int32),
      scratch_types=(
          [pltpu.VMEM((per_w,), jnp.int32)]
          + [pltpu.VMEM((CHUNK, D), jnp.int32)] * nslot
          + [pltpu.SemaphoreType.DMA] * (2 * nslot)
      ),
  )(_gather_body)


BA = 200
ISCALE = 1.0 / (HD ** 0.5)
SLICES = 5


def _attn_body(q_ref, kvg_ref, hm_ref, hmt_ref, wo_ref, bo_ref, o_ref):
  q = q_ref[...]
  kvg = kvg_ref[...]
  kg = lax.bitcast_convert_type(kvg << 16, jnp.float32)
  vg = lax.bitcast_convert_type(kvg & jnp.int32(-65536), jnp.float32)
  prod = (kg.reshape(BA, DEG, D) * q[:, None, :]).reshape(BA * DEG, D)
  sim = jnp.dot(prod, hm_ref[...], preferred_element_type=jnp.float32)
  sim = (sim * ISCALE).reshape(BA, DEG, D)
  m = jnp.max(sim, axis=1, keepdims=True)
  p = jnp.exp(sim - m)
  s = jnp.sum(p, axis=1, keepdims=True)
  attn = (p / s).reshape(BA * DEG, D)
  aw = jnp.dot(attn, hmt_ref[...], preferred_element_type=jnp.float32)
  ov = (aw * vg).reshape(BA, DEG, D).sum(axis=1)
  o_ref[...] = jnp.dot(ov, wo_ref[...], preferred_element_type=jnp.float32) + bo_ref[...]


def _attention(q, kvg, hm, hmt, wot, bo2, n_nodes, node_off):
  full = lambda i: (0, 0)
  off_blk = node_off // BA
  return pl.pallas_call(
      _attn_body,
      grid=(n_nodes // BA,),
      in_specs=[
          pl.BlockSpec((BA, D), lambda i: (i + off_blk, 0)),
          pl.BlockSpec((BA * DEG, D), lambda i: (i, 0)),
          pl.BlockSpec((D, D), full),
          pl.BlockSpec((D, D), full),
          pl.BlockSpec((D, D), full),
          pl.BlockSpec((1, D), full),
      ],
      out_specs=pl.BlockSpec((BA, D), lambda i: (i, 0)),
      out_shape=jax.ShapeDtypeStruct((n_nodes, D), jnp.float32),
  )(q, kvg, hm, hmt, wot, bo2)


def kernel(feats, edge_index, edge_attr, Wq, bq, Wk, bk, Wv, bv, Wo, bo):
  del edge_attr
  q, kv = _project(feats, Wq.T, Wk.T, Wv.T,
                   bq.reshape(1, D), bk.reshape(1, D), bv.reshape(1, D))
  src = edge_index[:, 0]
  d_ids = jnp.arange(D, dtype=jnp.int32)
  hm = (d_ids[:, None] // HD == d_ids[None, :]).astype(jnp.float32)
  hmt = (d_ids[:, None] == d_ids[None, :] // HD).astype(jnp.float32)
  wot = Wo.T
  bo2 = bo.reshape(1, D)
  ns = N // SLICES
  es = E // SLICES
  gather = _make_gather(es, 10)
  kvgs = [gather(kv, lax.slice_in_dim(src, i * es, (i + 1) * es))
          for i in range(SLICES)]
  outs = [_attention(q, kvgs[i], hm, hmt, wot, bo2, ns, i * ns)
          for i in range(SLICES)]
  return jnp.concatenate(outs, axis=0)
